# baseline (speedup 1.0000x reference)
import functools

import jax
import jax.numpy as jnp
from jax.experimental import pallas as pl
from jax.experimental.pallas import tpu as pltpu

_VARIANCES = (0.1, 0.2)
_THRESHOLD = 0.5
_NEGPOS_RATIO = 3

_NP = 8732
_R, _C = 72, 128
_PAD = _R * _C - _NP
_NB = 9
_BR = _R // _NB


def _match_body(truths_ref, labels_ref, priors_ref, pos_ref, btid_ref,
                btov, btidx, pmax, pminidx, num, num_objs):
    sl = pl.program_id(0)

    pcx = priors_ref[0, pl.ds(sl * _BR, _BR), :]
    pcy = priors_ref[1, pl.ds(sl * _BR, _BR), :]
    pw = priors_ref[2, pl.ds(sl * _BR, _BR), :]
    ph = priors_ref[3, pl.ds(sl * _BR, _BR), :]
    px1 = pcx - pw / 2.0
    py1 = pcy - ph / 2.0
    px2 = pcx + pw / 2.0
    py2 = pcy + ph / 2.0
    parea = (px2 - px1) * (py2 - py1)

    idx_blk = (sl * _BR * _C
               + jax.lax.broadcasted_iota(jnp.int32, (_BR, _C), 0) * _C
               + jax.lax.broadcasted_iota(jnp.int32, (_BR, _C), 1))

    bt_ov = jnp.full((num, _BR, _C), -1.0, dtype=jnp.float32)
    bt_id = jnp.zeros((num, _BR, _C), dtype=jnp.int32)
    for j in range(num_objs):
        tx1 = truths_ref[:, 0, j][:, None, None]
        ty1 = truths_ref[:, 1, j][:, None, None]
        tx2 = truths_ref[:, 2, j][:, None, None]
        ty2 = truths_ref[:, 3, j][:, None, None]
        ix = jnp.clip(jnp.minimum(px2, tx2) - jnp.maximum(px1, tx1),
                      0.0, None)
        iy = jnp.clip(jnp.minimum(py2, ty2) - jnp.maximum(py1, ty1),
                      0.0, None)
        inter = ix * iy
        tarea = (tx2 - tx1) * (ty2 - ty1)
        ov = inter / (tarea + parea - inter)
        upd = ov > bt_ov
        bt_ov = jnp.where(upd, ov, bt_ov)
        bt_id = jnp.where(upd, j, bt_id)
        mj = jnp.max(ov, axis=(1, 2))
        mn = jnp.min(jnp.where(ov == mj[:, None, None], idx_blk,
                               jnp.int32(2**30)), axis=(1, 2))
        pmax[sl, j, :] = mj
        pminidx[sl, j, :] = mn
    btov[:, pl.ds(sl * _BR, _BR), :] = bt_ov
    btidx[:, pl.ds(sl * _BR, _BR), :] = bt_id

    @pl.when(sl == _NB - 1)
    def _finalize():
        pm = pmax[...]
        pi = pminidx[...]
        gmax = jnp.max(pm, axis=0)
        bp = jnp.min(jnp.where(pm == gmax[None], pi, jnp.int32(2**30)),
                     axis=0)

        idx = (jax.lax.broadcasted_iota(jnp.int32, (_R, _C), 0) * _C
               + jax.lax.broadcasted_iota(jnp.int32, (_R, _C), 1))
        b_ov = btov[...]
        b_id = btidx[...]
        for j in range(num_objs):
            eq = idx[None] == bp[j, :][:, None, None]
            b_ov = jnp.where(eq, 2.0, b_ov)
            b_id = jnp.where(eq, j, b_id)

        conf_t = jnp.zeros((num, _R, _C), dtype=jnp.float32)
        for j in range(num_objs):
            conf_t = jnp.where(b_id == j, labels_ref[:, j][:, None, None],
                               conf_t)
        conf_t = jnp.where(b_ov < _THRESHOLD, 0.0, conf_t)
        pos = (conf_t > 0.0) & (idx[None] < _NP)
        pos_ref[...] = pos.astype(jnp.float32)
        btid_ref[...] = b_id


def _loss_body(truths_ref, priors_ref, loc_ref, conf_ref, pos_ref, btid_ref,
               out_ref, vA, part, num, num_objs, num_classes):
    i = pl.program_id(0)

    @pl.when(i < _NB)
    def _phase1():
        sl = i
        x = conf_ref[0]
        s = jnp.sum(jnp.exp(x), axis=1)
        lse = jnp.log(s)

        pos = pos_ref[:, pl.ds(sl * _BR, _BR), :] > 0.0
        bt_id = btid_ref[:, pl.ds(sl * _BR, _BR), :]
        idx_blk = (sl * _BR * _C
                   + jax.lax.broadcasted_iota(jnp.int32, (_BR, _C), 0) * _C
                   + jax.lax.broadcasted_iota(jnp.int32, (_BR, _C), 1))

        part[0, sl, :] = jnp.sum(pos.astype(jnp.float32), axis=(1, 2))
        part[2, sl, :] = jnp.sum(jnp.where(pos, lse - x[:, 1], 0.0),
                                 axis=(1, 2))
        vA[:, pl.ds(sl * _BR, _BR), :] = jnp.maximum(
            jnp.where(pos | (idx_blk[None] >= _NP), 0.0, lse - x[:, 0]), 0.0)

        pcx = priors_ref[0, pl.ds(sl * _BR, _BR), :]
        pcy = priors_ref[1, pl.ds(sl * _BR, _BR), :]
        pw = priors_ref[2, pl.ds(sl * _BR, _BR), :]
        ph = priors_ref[3, pl.ds(sl * _BR, _BR), :]

        mt = []
        for c in range(4):
            acc = jnp.zeros((num, _BR, _C), dtype=jnp.float32)
            for j in range(num_objs):
                acc = jnp.where(bt_id == j, truths_ref[:, c, j][:, None, None],
                                acc)
            mt.append(acc)
        mx1, my1, mx2, my2 = mt
        g = (((mx1 + mx2) / 2.0 - pcx) / (_VARIANCES[0] * pw),
             ((my1 + my2) / 2.0 - pcy) / (_VARIANCES[0] * ph),
             jnp.log((mx2 - mx1) / pw) / _VARIANCES[1],
             jnp.log((my2 - my1) / ph) / _VARIANCES[1])
        ll = jnp.zeros((num,), dtype=jnp.float32)
        for c in range(4):
            d = loc_ref[0, :, c] - g[c]
            ad = jnp.abs(d)
            sl1 = jnp.where(ad < 1.0, 0.5 * d * d, ad - 0.5)
            ll += jnp.sum(jnp.where(pos, sl1, 0.0), axis=(1, 2))
        part[1, sl, :] = ll

    @pl.when(i == _NB)
    def _phase2():
        npos = jnp.sum(part[0], axis=0)
        ll_tot = jnp.sum(part[1])
        ce_pos = jnp.sum(part[2], axis=0)

        k = jnp.minimum((_NEGPOS_RATIO * npos).astype(jnp.int32),
                        jnp.int32(_NP - 1))
        v = vA[...]
        vb = jax.lax.bitcast_convert_type(v, jnp.int32)
        t = jnp.zeros((num,), dtype=jnp.int32)
        for b in range(30, 12, -1):
            cand = t | jnp.int32(1 << b)
            cnt = jnp.sum((vb >= cand[:, None, None]).astype(jnp.int32),
                          axis=(1, 2))
            t = jnp.where(cnt >= k, cand, t)
        cnt_gt = jnp.sum((vb > t[:, None, None]).astype(jnp.int32),
                         axis=(1, 2))
        sum_gt = jnp.sum(jnp.where(vb > t[:, None, None], v, 0.0),
                         axis=(1, 2))
        tval = jax.lax.bitcast_convert_type(t, jnp.float32)
        topk = sum_gt + (k - cnt_gt).astype(jnp.float32) * tval
        topk = jnp.where(k > 0, topk, 0.0)
        lc_tot = jnp.sum(ce_pos + topk)
        n = jnp.sum(npos)

        lane = jax.lax.broadcasted_iota(jnp.int32, (1, 128), 1)
        out_ref[...] = (jnp.where(lane == 0, ll_tot / n, 0.0)
                        + jnp.where(lane == 1, lc_tot / n, 0.0))


@jax.jit
def kernel(loc_data, conf_data, priors, targets):
    num, num_priors, num_classes = conf_data.shape
    num_objs = targets.shape[1]

    conf_p = jnp.pad(jnp.transpose(conf_data, (0, 2, 1)),
                     ((0, 0), (0, 0), (0, _PAD)))
    conf_p = jnp.transpose(
        conf_p.reshape(num, num_classes, _NB, _BR, _C), (2, 0, 1, 3, 4))
    loc_p = jnp.pad(loc_data, ((0, 0), (0, _PAD), (0, 0)))
    loc_p = jnp.transpose(
        loc_p.reshape(num, _NB, _BR * _C, 4), (1, 0, 3, 2)).reshape(
            _NB, num, 4, _BR, _C)
    priors_p = jnp.pad(priors.T, ((0, 0), (0, _PAD))).reshape(4, _R, _C)
    truths = jnp.transpose(targets[:, :, :4], (0, 2, 1))
    labels = targets[:, :, 4]

    mbody = functools.partial(_match_body, num=num, num_objs=num_objs)
    pos_f, btid = pl.pallas_call(
        mbody,
        grid=(_NB,),
        in_specs=[
            pl.BlockSpec((num, 4, num_objs), lambda i: (0, 0, 0)),
            pl.BlockSpec((num, num_objs), lambda i: (0, 0)),
            pl.BlockSpec((4, _R, _C), lambda i: (0, 0, 0)),
        ],
        out_specs=[
            pl.BlockSpec((num, _R, _C), lambda i: (0, 0, 0)),
            pl.BlockSpec((num, _R, _C), lambda i: (0, 0, 0)),
        ],
        out_shape=[
            jax.ShapeDtypeStruct((num, _R, _C), jnp.float32),
            jax.ShapeDtypeStruct((num, _R, _C), jnp.int32),
        ],
        scratch_shapes=[
            pltpu.VMEM((num, _R, _C), jnp.float32),
            pltpu.VMEM((num, _R, _C), jnp.int32),
            pltpu.VMEM((_NB, num_objs, num), jnp.float32),
            pltpu.VMEM((_NB, num_objs, num), jnp.int32),
        ],
    )(truths, labels, priors_p)

    lbody = functools.partial(_loss_body, num=num, num_objs=num_objs,
                              num_classes=num_classes)
    out = pl.pallas_call(
        lbody,
        grid=(_NB + 1,),
        in_specs=[
            pl.BlockSpec((num, 4, num_objs), lambda i: (0, 0, 0)),
            pl.BlockSpec((4, _R, _C), lambda i: (0, 0, 0)),
            pl.BlockSpec((1, num, 4, _BR, _C),
                         lambda i: (jnp.clip(i, 0, _NB - 1), 0, 0, 0, 0)),
            pl.BlockSpec((1, num, num_classes, _BR, _C),
                         lambda i: (jnp.clip(i, 0, _NB - 1), 0, 0, 0, 0)),
            pl.BlockSpec((num, _R, _C), lambda i: (0, 0, 0)),
            pl.BlockSpec((num, _R, _C), lambda i: (0, 0, 0)),
        ],
        out_specs=pl.BlockSpec((1, 128), lambda i: (0, 0)),
        out_shape=jax.ShapeDtypeStruct((1, 128), jnp.float32),
        scratch_shapes=[
            pltpu.VMEM((num, _R, _C), jnp.float32),
            pltpu.VMEM((3, _NB, num), jnp.float32),
        ],
    )(truths, priors_p, loc_p, conf_p, pos_f, btid)
    return (out[0, 0], out[0, 1])
